# Initial kernel scaffold; baseline (speedup 1.0000x reference)
import jax
import jax.numpy as jnp
from jax.experimental import pallas as pl


def kernel(input, weight, grid):
    raise NotImplementedError("write your pallas kernel here")


# speedup vs baseline: 545.2893x; 545.2893x over previous
import jax
import jax.numpy as jnp
from jax.experimental import pallas as pl

_M0 = 0.3
_ITERS = 28


def _dtm_tc_kernel(x_ref, w_ref, g_ref, o_ref):
    B = x_ref.shape[0]
    Mt = g_ref.shape[0]
    gx = g_ref[:, 0:1]
    gy = g_ref[:, 1:2]
    for b in range(B):
        xx = x_ref[b, 0:1, :]
        xy = x_ref[b, 1:2, :]
        w = w_ref[b]
        wb = _M0 * jnp.sum(w)
        dx = gx - xx
        dy = gy - xy
        d2 = jnp.maximum(dx * dx + dy * dy, 1e-12)
        wd2 = w * d2
        lo = jnp.zeros((Mt, 1), jnp.float32)
        hi = jnp.full((Mt, 1), 8.5, jnp.float32)

        def body(_, carry):
            lo, hi = carry
            mid = 0.5 * (lo + hi)
            wsum = jnp.sum(jnp.where(d2 < mid, w, 0.0), axis=1, keepdims=True)
            pred = wsum < wb
            return jnp.where(pred, mid, lo), jnp.where(pred, hi, mid)

        lo, hi = jax.lax.fori_loop(0, _ITERS, body, (lo, hi))
        tau = 0.5 * (lo + hi)
        mask = d2 < tau
        wl = jnp.sum(jnp.where(mask, w, 0.0), axis=1, keepdims=True)
        sl = jnp.sum(jnp.where(mask, wd2, 0.0), axis=1, keepdims=True)
        o_ref[:, b : b + 1] = jnp.sqrt((sl + tau * (wb - wl)) / wb)


def kernel(input, weight, grid):
    B, N, _ = input.shape
    M = grid.shape[0]
    Mt = 256
    x_t = jnp.swapaxes(input, 1, 2)
    w3 = weight[:, None, :]
    out = pl.pallas_call(
        _dtm_tc_kernel,
        grid=(M // Mt,),
        in_specs=[
            pl.BlockSpec((B, 2, N), lambda m: (0, 0, 0)),
            pl.BlockSpec((B, 1, N), lambda m: (0, 0, 0)),
            pl.BlockSpec((Mt, 2), lambda m: (m, 0)),
        ],
        out_specs=pl.BlockSpec((Mt, B), lambda m: (m, 0)),
        out_shape=jax.ShapeDtypeStruct((M, B), jnp.float32),
    )(x_t, w3, grid)
    return out.T


# speedup vs baseline: 852.7444x; 1.5638x over previous
import jax
import jax.numpy as jnp
from jax.experimental import pallas as pl

_M0 = 0.3
_ITERS = 16


def _dtm_tc_kernel(x_ref, w_ref, g_ref, o_ref):
    B = x_ref.shape[0]
    Mt = g_ref.shape[0]
    gx = g_ref[:, 0:1]
    gy = g_ref[:, 1:2]
    for b in range(B):
        xx = x_ref[b, 0:1, :]
        xy = x_ref[b, 1:2, :]
        w = w_ref[b]
        wb = _M0 * jnp.sum(w)
        dx = gx - xx
        dy = gy - xy
        d2 = jnp.maximum(dx * dx + dy * dy, 1e-12)
        wd2 = w * d2
        lo = jnp.zeros((Mt, 1), jnp.float32)
        hi = jnp.full((Mt, 1), 8.5, jnp.float32)

        def body(_, carry):
            lo, hi = carry
            mid = 0.5 * (lo + hi)
            wsum = jnp.sum(jnp.where(d2 < mid, w, 0.0), axis=1, keepdims=True)
            pred = wsum < wb
            return jnp.where(pred, mid, lo), jnp.where(pred, hi, mid)

        lo, hi = jax.lax.fori_loop(0, _ITERS, body, (lo, hi))
        tau = 0.5 * (lo + hi)
        mask = d2 < tau
        wl = jnp.sum(jnp.where(mask, w, 0.0), axis=1, keepdims=True)
        sl = jnp.sum(jnp.where(mask, wd2, 0.0), axis=1, keepdims=True)
        o_ref[:, b : b + 1] = jnp.sqrt((sl + tau * (wb - wl)) / wb)


def kernel(input, weight, grid):
    B, N, _ = input.shape
    M = grid.shape[0]
    Mt = 256
    x_t = jnp.swapaxes(input, 1, 2)
    w3 = weight[:, None, :]
    out = pl.pallas_call(
        _dtm_tc_kernel,
        grid=(M // Mt,),
        in_specs=[
            pl.BlockSpec((B, 2, N), lambda m: (0, 0, 0)),
            pl.BlockSpec((B, 1, N), lambda m: (0, 0, 0)),
            pl.BlockSpec((Mt, 2), lambda m: (m, 0)),
        ],
        out_specs=pl.BlockSpec((Mt, B), lambda m: (m, 0)),
        out_shape=jax.ShapeDtypeStruct((M, B), jnp.float32),
    )(x_t, w3, grid)
    return out.T
